# Initial kernel scaffold; baseline (speedup 1.0000x reference)
import functools

import jax
import jax.numpy as jnp
from jax import lax
from jax.experimental import pallas as pl
from jax.experimental.pallas import tpu as pltpu
from jax.experimental.pallas import tpu_sc as plsc

N_NODES = 10000
N_EDGES = 320000
F = 128
NC = 2
NS = 16
NW = NC * NS
EPW = N_EDGES // NW
CH = 80
NCHUNK = EPW // CH
RPS = N_NODES // NS
WB = 125
CNTW = 16

TB = 1000


def _tc1_body(x_ref, win_ref, bin_ref, w0_ref, h_ref, t0_ref):
    h = jnp.maximum(
        jnp.dot(x_ref[...], win_ref[...], preferred_element_type=jnp.float32)
        + bin_ref[...], 0.0)
    h_ref[...] = h
    t0_ref[...] = jnp.dot(h, w0_ref[...], preferred_element_type=jnp.float32)


def _tc1(x, W_in, b_in, W0):
    grid = (N_NODES // TB,)
    return pl.pallas_call(
        _tc1_body,
        grid=grid,
        in_specs=[
            pl.BlockSpec((TB, F), lambda i: (i, 0)),
            pl.BlockSpec((F, F), lambda i: (0, 0)),
            pl.BlockSpec((1, F), lambda i: (0, 0)),
            pl.BlockSpec((F, F), lambda i: (0, 0)),
        ],
        out_specs=[
            pl.BlockSpec((TB, F), lambda i: (i, 0)),
            pl.BlockSpec((TB, F), lambda i: (i, 0)),
        ],
        out_shape=[
            jax.ShapeDtypeStruct((N_NODES, F), jnp.float32),
            jax.ShapeDtypeStruct((N_NODES, F), jnp.float32),
        ],
    )(x, W_in, b_in, W0)


def _tc2_body(p_ref, c_ref, h_ref, b0_ref, w1_ref, t1_ref):
    acc = p_ref[0] + p_ref[1]
    cnt = c_ref[0] + c_ref[1]
    count = jnp.maximum(cnt[:, 0:1], 1.0)
    m = jnp.maximum(acc / count + b0_ref[...], 0.0)
    t1_ref[...] = jnp.dot(m + h_ref[...], w1_ref[...],
                          preferred_element_type=jnp.float32)


def _tc2(parts, cnts, h, b0, W1):
    grid = (N_NODES // TB,)
    return pl.pallas_call(
        _tc2_body,
        grid=grid,
        in_specs=[
            pl.BlockSpec((NC, TB, F), lambda i: (0, i, 0)),
            pl.BlockSpec((NC, TB, CNTW), lambda i: (0, i, 0)),
            pl.BlockSpec((TB, F), lambda i: (i, 0)),
            pl.BlockSpec((1, F), lambda i: (0, 0)),
            pl.BlockSpec((F, F), lambda i: (0, 0)),
        ],
        out_specs=pl.BlockSpec((TB, F), lambda i: (i, 0)),
        out_shape=jax.ShapeDtypeStruct((N_NODES, F), jnp.float32),
    )(parts, cnts, h, b0, W1)


def _tc3_body(p_ref, c_ref, b1_ref, o_ref):
    acc = p_ref[0] + p_ref[1]
    cnt = c_ref[0] + c_ref[1]
    count = jnp.maximum(cnt[:, 0:1], 1.0)
    m = acc / count + b1_ref[...]
    nrm = jnp.sqrt(jnp.sum(m * m, axis=1, keepdims=True))
    o_ref[...] = m / jnp.maximum(nrm, 1e-12)


def _tc3(parts, cnts, b1):
    grid = (N_NODES // TB,)
    return pl.pallas_call(
        _tc3_body,
        grid=grid,
        in_specs=[
            pl.BlockSpec((NC, TB, F), lambda i: (0, i, 0)),
            pl.BlockSpec((NC, TB, CNTW), lambda i: (0, i, 0)),
            pl.BlockSpec((1, F), lambda i: (0, 0)),
        ],
        out_specs=pl.BlockSpec((TB, F), lambda i: (i, 0)),
        out_shape=jax.ShapeDtypeStruct((N_NODES, F), jnp.float32),
    )(parts, cnts, b1)


def _sc_body(with_count, *refs):
    if with_count:
        (t_hbm, src_hbm, dst_hbm, out_hbm, cnt_hbm,
         srcv, dstv, rows, bounce, acc, sem, onesb, cbuf, cnt_sh) = refs
    else:
        (t_hbm, src_hbm, dst_hbm, out_hbm,
         srcv, dstv, rows, bounce, acc, sem) = refs

    cid = lax.axis_index("c")
    sid = lax.axis_index("s")
    wid = cid * NS + sid

    zeros16 = jnp.zeros((16,), jnp.float32)

    @pl.loop(0, WB)
    def _(r):
        for j in range(F // 16):
            bounce[r, pl.ds(j * 16, 16)] = zeros16

    @pl.loop(0, RPS // WB)
    def _(kk):
        pltpu.sync_copy(bounce, acc.at[pl.ds(sid * RPS + kk * WB, WB)])

    if with_count:
        ones16 = jnp.full((16,), 1.0, jnp.float32)

        @pl.loop(0, RPS)
        def _(r):
            cbuf[r, :] = zeros16

        @pl.loop(0, CH)
        def _(r):
            onesb[r, :] = ones16

        pltpu.sync_copy(cbuf, cnt_sh.at[pl.ds(sid * RPS, RPS)])

    plsc.subcore_barrier()

    base = wid * EPW

    @pl.loop(0, NCHUNK)
    def _(k):
        off = base + k * CH
        pltpu.sync_copy(src_hbm.at[pl.ds(off, CH)], srcv)
        pltpu.async_copy(t_hbm.at[srcv], rows, sem).wait()
        pltpu.sync_copy(dst_hbm.at[pl.ds(off, CH)], dstv)
        pltpu.sync_copy(rows, acc.at[dstv], add=True)
        if with_count:
            pltpu.sync_copy(onesb, cnt_sh.at[dstv], add=True)

    plsc.subcore_barrier()

    @pl.loop(0, RPS // WB)
    def _(kk):
        rr = sid * RPS + kk * WB
        pltpu.sync_copy(acc.at[pl.ds(rr, WB)], bounce)
        pltpu.sync_copy(bounce, out_hbm.at[pl.ds(cid * N_NODES + rr, WB)])

    if with_count:
        r0 = sid * RPS
        pltpu.sync_copy(cnt_sh.at[pl.ds(r0, RPS)], cbuf)
        pltpu.sync_copy(cbuf, cnt_hbm.at[pl.ds(cid * N_NODES + r0, RPS)])


def _sc_scatter(t, src, dst, with_count):
    mesh = plsc.VectorSubcoreMesh(core_axis_name="c", subcore_axis_name="s")
    out_type = [jax.ShapeDtypeStruct((NC * N_NODES, F), jnp.float32)]
    scratch = [
        pltpu.VMEM((CH,), jnp.int32),
        pltpu.VMEM((CH,), jnp.int32),
        pltpu.VMEM((CH, F), jnp.float32),
        pltpu.VMEM((WB, F), jnp.float32),
        pltpu.VMEM_SHARED((N_NODES, F), jnp.float32),
        pltpu.SemaphoreType.DMA,
    ]
    if with_count:
        out_type.append(jax.ShapeDtypeStruct((NC * N_NODES, CNTW), jnp.float32))
        scratch += [
            pltpu.VMEM((CH, CNTW), jnp.float32),
            pltpu.VMEM((RPS, CNTW), jnp.float32),
            pltpu.VMEM_SHARED((N_NODES, CNTW), jnp.float32),
        ]
    fn = pl.kernel(
        functools.partial(_sc_body, with_count),
        out_type=out_type,
        mesh=mesh,
        scratch_types=scratch,
    )
    return fn(t, src, dst)


def kernel(x, edge_index, W_in, b_in, W0, b0, W1, b1):
    src = edge_index[0]
    dst = edge_index[1]
    b_in2 = b_in.reshape(1, F)
    b02 = b0.reshape(1, F)
    b12 = b1.reshape(1, F)

    h, t0 = _tc1(x, W_in, b_in2, W0)

    part0_flat, cnt_flat = _sc_scatter(t0, src, dst, with_count=True)
    part0 = part0_flat.reshape(NC, N_NODES, F)
    cnts = cnt_flat.reshape(NC, N_NODES, CNTW)

    t1 = _tc2(part0, cnts, h, b02, W1)

    (part1_flat,) = _sc_scatter(t1, src, dst, with_count=False)
    part1 = part1_flat.reshape(NC, N_NODES, F)

    return _tc3(part1, cnts, b12)


# speedup vs baseline: 4.9862x; 4.9862x over previous
import functools

import jax
import jax.numpy as jnp
from jax import lax
from jax.experimental import pallas as pl
from jax.experimental.pallas import tpu as pltpu
from jax.experimental.pallas import tpu_sc as plsc

N_NODES = 10000
N_EDGES = 320000
F = 128
NC = 2
NS = 16
NW = NC * NS
EPW = N_EDGES // NW
CH = 80
NCHUNK = EPW // CH
N_PAD = 10240
RPS = N_PAD // NS
WB = 128
CNTW = 16

TB = 1000


def _tc1_body(x_ref, win_ref, bin_ref, w0_ref, h_ref, t0_ref):
    h = jnp.maximum(
        jnp.dot(x_ref[...], win_ref[...], preferred_element_type=jnp.float32)
        + bin_ref[...], 0.0)
    h_ref[...] = h
    t0_ref[...] = jnp.dot(h, w0_ref[...], preferred_element_type=jnp.float32)


def _tc1(x, W_in, b_in, W0):
    grid = (N_NODES // TB,)
    return pl.pallas_call(
        _tc1_body,
        grid=grid,
        in_specs=[
            pl.BlockSpec((TB, F), lambda i: (i, 0)),
            pl.BlockSpec((F, F), lambda i: (0, 0)),
            pl.BlockSpec((1, F), lambda i: (0, 0)),
            pl.BlockSpec((F, F), lambda i: (0, 0)),
        ],
        out_specs=[
            pl.BlockSpec((TB, F), lambda i: (i, 0)),
            pl.BlockSpec((TB, F), lambda i: (i, 0)),
        ],
        out_shape=[
            jax.ShapeDtypeStruct((N_NODES, F), jnp.float32),
            jax.ShapeDtypeStruct((N_NODES, F), jnp.float32),
        ],
    )(x, W_in, b_in, W0)


def _tc2_body(p_ref, c_ref, h_ref, b0_ref, w1_ref, t1_ref):
    acc = p_ref[0] + p_ref[1]
    cnt = c_ref[0] + c_ref[1]
    count = jnp.maximum(cnt[:, 0:1], 1.0)
    m = jnp.maximum(acc / count + b0_ref[...], 0.0)
    t1_ref[...] = jnp.dot(m + h_ref[...], w1_ref[...],
                          preferred_element_type=jnp.float32)


def _tc2(parts, cnts, h, b0, W1):
    grid = (N_NODES // TB,)
    return pl.pallas_call(
        _tc2_body,
        grid=grid,
        in_specs=[
            pl.BlockSpec((NC, TB, F), lambda i: (0, i, 0)),
            pl.BlockSpec((NC, TB, CNTW), lambda i: (0, i, 0)),
            pl.BlockSpec((TB, F), lambda i: (i, 0)),
            pl.BlockSpec((1, F), lambda i: (0, 0)),
            pl.BlockSpec((F, F), lambda i: (0, 0)),
        ],
        out_specs=pl.BlockSpec((TB, F), lambda i: (i, 0)),
        out_shape=jax.ShapeDtypeStruct((N_NODES, F), jnp.float32),
    )(parts, cnts, h, b0, W1)


def _tc3_body(p_ref, c_ref, b1_ref, o_ref):
    acc = p_ref[0] + p_ref[1]
    cnt = c_ref[0] + c_ref[1]
    count = jnp.maximum(cnt[:, 0:1], 1.0)
    m = acc / count + b1_ref[...]
    nrm = jnp.sqrt(jnp.sum(m * m, axis=1, keepdims=True))
    o_ref[...] = m / jnp.maximum(nrm, 1e-12)


def _tc3(parts, cnts, b1):
    grid = (N_NODES // TB,)
    return pl.pallas_call(
        _tc3_body,
        grid=grid,
        in_specs=[
            pl.BlockSpec((NC, TB, F), lambda i: (0, i, 0)),
            pl.BlockSpec((NC, TB, CNTW), lambda i: (0, i, 0)),
            pl.BlockSpec((1, F), lambda i: (0, 0)),
        ],
        out_specs=pl.BlockSpec((TB, F), lambda i: (i, 0)),
        out_shape=jax.ShapeDtypeStruct((N_NODES, F), jnp.float32),
    )(parts, cnts, b1)


def _sc_body(with_count, *refs):
    if with_count:
        (t_hbm, src_hbm, dst_hbm, out_hbm, cnt_hbm,
         srcv, dstv, rows, bounce, acc, sem, onesb, cbuf, cnt_sh) = refs
    else:
        (t_hbm, src_hbm, dst_hbm, out_hbm,
         srcv, dstv, rows, bounce, acc, sem) = refs

    cid = lax.axis_index("c")
    sid = lax.axis_index("s")
    wid = cid * NS + sid

    zeros16 = jnp.zeros((16,), jnp.float32)

    @pl.loop(0, WB)
    def _(r):
        for j in range(F // 16):
            bounce[r, pl.ds(j * 16, 16)] = zeros16

    @pl.loop(0, RPS // WB)
    def _(kk):
        pltpu.sync_copy(bounce, acc.at[pl.ds(sid * RPS + kk * WB, WB)])

    if with_count:
        ones16 = jnp.full((16,), 1.0, jnp.float32)

        @pl.loop(0, RPS)
        def _(r):
            cbuf[r, :] = zeros16

        @pl.loop(0, CH)
        def _(r):
            onesb[r, :] = ones16

        pltpu.sync_copy(cbuf, cnt_sh.at[pl.ds(sid * RPS, RPS)])

    plsc.subcore_barrier()

    base = wid * EPW

    @pl.loop(0, NCHUNK)
    def _(k):
        off = base + k * CH
        pltpu.sync_copy(src_hbm.at[pl.ds(off, CH)], srcv)
        pltpu.async_copy(t_hbm.at[srcv], rows, sem).wait()
        pltpu.sync_copy(dst_hbm.at[pl.ds(off, CH)], dstv)
        pltpu.sync_copy(rows, acc.at[dstv], add=True)
        if with_count:
            pltpu.sync_copy(onesb, cnt_sh.at[dstv], add=True)

    plsc.subcore_barrier()

    @pl.loop(0, RPS // WB)
    def _(kk):
        rr = sid * RPS + kk * WB
        pltpu.sync_copy(acc.at[pl.ds(rr, WB)], bounce)
        pltpu.sync_copy(bounce, out_hbm.at[pl.ds(cid * N_PAD + rr, WB)])

    if with_count:
        r0 = sid * RPS
        pltpu.sync_copy(cnt_sh.at[pl.ds(r0, RPS)], cbuf)
        pltpu.sync_copy(cbuf, cnt_hbm.at[pl.ds(cid * N_PAD + r0, RPS)])


def _sc_scatter(t, src, dst, with_count):
    mesh = plsc.VectorSubcoreMesh(core_axis_name="c", subcore_axis_name="s")
    out_type = [jax.ShapeDtypeStruct((NC * N_PAD, F), jnp.float32)]
    scratch = [
        pltpu.VMEM((CH,), jnp.int32),
        pltpu.VMEM((CH,), jnp.int32),
        pltpu.VMEM((CH, F), jnp.float32),
        pltpu.VMEM((WB, F), jnp.float32),
        pltpu.VMEM_SHARED((N_PAD, F), jnp.float32),
        pltpu.SemaphoreType.DMA,
    ]
    if with_count:
        out_type.append(jax.ShapeDtypeStruct((NC * N_PAD, CNTW), jnp.float32))
        scratch += [
            pltpu.VMEM((CH, CNTW), jnp.float32),
            pltpu.VMEM((RPS, CNTW), jnp.float32),
            pltpu.VMEM_SHARED((N_PAD, CNTW), jnp.float32),
        ]
    fn = pl.kernel(
        functools.partial(_sc_body, with_count),
        out_type=out_type,
        mesh=mesh,
        scratch_types=scratch,
        compiler_params=pltpu.CompilerParams(use_tc_tiling_on_sc=False),
    )
    return fn(t, src, dst)


def kernel(x, edge_index, W_in, b_in, W0, b0, W1, b1):
    src = edge_index[0]
    dst = edge_index[1]
    b_in2 = b_in.reshape(1, F)
    b02 = b0.reshape(1, F)
    b12 = b1.reshape(1, F)

    h, t0 = _tc1(x, W_in, b_in2, W0)

    part0_flat, cnt_flat = _sc_scatter(t0, src, dst, with_count=True)
    part0 = part0_flat.reshape(NC, N_PAD, F)
    cnts = cnt_flat.reshape(NC, N_PAD, CNTW)

    t1 = _tc2(part0, cnts, h, b02, W1)

    (part1_flat,) = _sc_scatter(t1, src, dst, with_count=False)
    part1 = part1_flat.reshape(NC, N_PAD, F)

    return _tc3(part1, cnts, b12)
